# baseline (speedup 1.0000x reference)
import functools

import jax
import jax.numpy as jnp
from jax import lax
from jax.experimental import pallas as pl
from jax.experimental.pallas import tpu as pltpu
from jax.experimental.pallas import tpu_sc as plsc

_RATIO = 3.0
_EPS = 1e-06

_B = 4
_NPOS = 512 * 512
_NTILES = 16
_PER_TILE = _NPOS // _NTILES
_CH = 8192
_NCH = _PER_TILE // _CH
_NB = 2048
_NB2 = 32
_L = 16
_HIST = _NB * _L
_ROW = _NB
_ROW2 = 128

_GATHER_DNUMS = lax.GatherDimensionNumbers(
    offset_dims=(), collapsed_slice_dims=(0,), start_index_map=(0,))


def _dg(v, idx):
    return lax.gather(v, idx[:, None], _GATHER_DNUMS, (1,),
                      mode=lax.GatherScatterMode.PROMISE_IN_BOUNDS)


_BH = 512
_G = 512 // _BH
_RB = _BH * 512 // 128


def _tc_body(pred_ref, t_ref, msk_ref, word_ref, stats_ref,
             sl_ref, sp_ref, macc_ref, lacc_ref):
    i = pl.program_id(0)
    j = pl.program_id(1)
    p = pred_ref[0]
    t = t_ref[0]
    mk = msk_ref[0]
    x = jnp.where(t > 0.5, p, 1.0 - p)
    loss = jnp.maximum(jnp.minimum(-jnp.log(x), 100.0), 0.0)
    pos = t * mk
    neg = (1.0 - t) * mk
    lacc_ref[j] = loss

    @pl.when(j == 0)
    def _():
        sl_ref[...] = loss
        sp_ref[...] = pos
        macc_ref[...] = neg

    @pl.when(j > 0)
    def _():
        sl_ref[...] = sl_ref[...] + loss
        sp_ref[...] = sp_ref[...] + pos
        macc_ref[...] = macc_ref[...] + neg

    @pl.when((i == 0) & (j == 0))
    def _():
        stats_ref[...] = jnp.zeros_like(stats_ref)

    @pl.when(j == _B - 1)
    def _():
        w = macc_ref[...].astype(jnp.int32)
        for jj in range(_B):
            bits = lax.bitcast_convert_type(lacc_ref[jj], jnp.int32)
            rnd = (bits + 0x8000) & jnp.int32(-65536)
            word_ref[pl.ds(jj * _RB, _RB), :] = (rnd | w).reshape(_RB, 128)
        stats_ref[0:1, :] = stats_ref[0:1, :] + jnp.sum(sp_ref[...])
        stats_ref[1:2, :] = stats_ref[1:2, :] + jnp.sum(macc_ref[...])
        stats_ref[2:3, :] = stats_ref[2:3, :] + jnp.sum(
            sl_ref[...] * sp_ref[...])


def _tc_stage(pred3, map3, mask3):
    in_spec = pl.BlockSpec((1, _BH, 512), lambda i, j: (j, i, 0))
    return pl.pallas_call(
        _tc_body,
        grid=(_G, _B),
        in_specs=[in_spec, in_spec, in_spec],
        out_specs=[
            pl.BlockSpec((_B * _RB, 128), lambda i, j: (i, 0)),
            pl.BlockSpec((8, 128), lambda i, j: (0, 0)),
        ],
        out_shape=[
            jax.ShapeDtypeStruct((_G * _B * _RB, 128), jnp.int32),
            jax.ShapeDtypeStruct((8, 128), jnp.float32),
        ],
        scratch_shapes=[
            pltpu.VMEM((_BH, 512), jnp.float32),
            pltpu.VMEM((_BH, 512), jnp.float32),
            pltpu.VMEM((_BH, 512), jnp.float32),
            pltpu.VMEM((_B, _BH, 512), jnp.float32),
        ],
    )(pred3, map3, mask3)


def _sc_body(word_hbm, stats_hbm, out_hbm,
             vbuf, hist_ref, red_ref, acc_ref,
             stats_buf, out_buf, shared, sem0, sem2):
    sid = lax.axis_index("s")
    cid = lax.axis_index("c")
    base = sid * (_B * _PER_TILE)

    _STRIPE = _B * _PER_TILE
    first_handles = [
        pltpu.async_copy(word_hbm.at[pl.ds(base + h * (_STRIPE // 4),
                                           _STRIPE // 4)],
                         vbuf.at[pl.ds(h * (_STRIPE // 4), _STRIPE // 4)],
                         sem0)
        for h in range(4)]
    pltpu.sync_copy(stats_hbm, stats_buf)
    pos_num = stats_buf[pl.ds(0, _L)]
    neg_count = stats_buf[pl.ds(128, _L)]
    pos_sum = stats_buf[pl.ds(256, _L)]
    k_sel = jnp.minimum(neg_count, pos_num * _RATIO)

    lane = lax.iota(jnp.int32, _L)
    zeros16 = jnp.zeros((_L,), jnp.float32)
    lane15 = jnp.full((_L,), _L - 1, jnp.int32)

    def splat_sum(v):
        for s in (1, 2, 4, 8):
            v = v + _dg(v, lane ^ s)
        return v

    def incl_prefix(v):
        r = v
        for s in (1, 2, 4, 8):
            g = _dg(r, jnp.maximum(lane - s, 0))
            r = r + jnp.where(lane >= s, g, zeros16)
        return r

    def zero_hist(nwords):
        @plsc.parallel_loop(0, nwords // _L, unroll=8)
        def _(i):
            hist_ref[pl.ds(i * _L, _L)] = zeros16

    def hist_pass(bstar_splat):
        @plsc.parallel_loop(0, _B * _PER_TILE // _L, unroll=4,
                            carry=zeros16)
        def vb(i, sa):
            bits = vbuf[pl.ds(i * _L, _L)]
            b1 = bits >> 21
            wf = (bits & 7).astype(jnp.float32)
            if bstar_splat is None:
                plsc.addupdate_scatter(hist_ref, [lane * _NB + b1], wf)
            else:
                sel = b1 == bstar_splat
                b2 = (bits >> 16) & (_NB2 - 1)
                idx = lane * _NB2 + b2
                v = lax.bitcast_convert_type(bits & jnp.int32(-65536),
                                             jnp.float32)
                wv = wf * v
                plsc.addupdate_scatter(hist_ref, [idx], wf, mask=sel)
                plsc.addupdate_scatter(hist_ref, [idx + _NB2 * _L], wv,
                                       mask=sel)
                sa = sa + jnp.where(b1 > bstar_splat, wv, zeros16)
            return sa
        return vb

    def lane_reduce(nbins, with_sums):
        @plsc.parallel_loop(0, nbins // _L, unroll=2)
        def _(i):
            cacc = zeros16
            for l in range(_L):
                cacc = cacc + hist_ref[pl.ds(l * nbins + i * _L, _L)]
            red_ref[pl.ds(i * _L, _L)] = cacc
            if with_sums:
                sacc = zeros16
                for l in range(_L):
                    sacc = sacc + hist_ref[
                        pl.ds(nbins * _L + l * nbins + i * _L, _L)]
                red_ref[pl.ds(nbins + i * _L, _L)] = sacc

    def merge(width):
        pltpu.sync_copy(red_ref.at[pl.ds(0, width)],
                        shared.at[sid, pl.ds(0, width)])
        plsc.subcore_barrier()
        hs = [pltpu.async_copy(shared.at[r, pl.ds(0, width)],
                               hist_ref.at[pl.ds(r * width, width)], sem2)
              for r in range(_NTILES)]
        for h in hs:
            h.wait()

        @plsc.parallel_loop(0, width // _L, unroll=2)
        def _(i):
            o = i * _L
            a = hist_ref[pl.ds(o, _L)]
            for r in range(1, _NTILES):
                a = a + hist_ref[pl.ds(r * width + o, _L)]
            acc_ref[pl.ds(o, _L)] = a
        plsc.subcore_barrier()

    def scan(ktarget, nbins, with_sums):
        def tb(i, tot):
            return tot + acc_ref[pl.ds(i * _L, _L)]
        total = splat_sum(lax.fori_loop(0, nbins // _L, tb, zeros16))
        thresh = total - ktarget

        def sb(i, carry):
            run, nm = carry
            v = acc_ref[pl.ds(i * _L, _L)]
            incl = incl_prefix(v)
            cb = run + (incl - v)
            nm = nm + jnp.where(cb <= thresh, 1.0, 0.0)
            return run + _dg(incl, lane15), nm
        _, nmask = lax.fori_loop(0, nbins // _L, sb, (zeros16, zeros16))
        bstar = (splat_sum(nmask) - 1.0).astype(jnp.int32)

        def ab(i, carry):
            ca, sa = carry
            msk = (lane + i * _L) > bstar
            ca = ca + jnp.where(msk, acc_ref[pl.ds(i * _L, _L)], zeros16)
            if with_sums:
                sa = sa + jnp.where(msk, acc_ref[pl.ds(nbins + i * _L, _L)],
                                    zeros16)
            return ca, sa
        ca, sa = lax.fori_loop(0, nbins // _L, ab, (zeros16, zeros16))
        return bstar, splat_sum(ca), splat_sum(sa)

    zero_hist(_HIST)
    for h in first_handles:
        h.wait()
    hist_pass(None)
    lane_reduce(_NB, False)
    merge(_ROW)
    bstar, cnt_a, _ = scan(k_sel, _NB, False)

    zero_hist(2 * _NB2 * _L)
    sa_vec = hist_pass(bstar)
    lane_reduce(_NB2, True)
    red_ref[pl.ds(2 * _NB2, _L)] = sa_vec
    merge(_ROW2)
    b2star, cnt_a2, sum_a2 = scan(k_sel - cnt_a, _NB2, True)
    sum_a = splat_sum(acc_ref[pl.ds(2 * _NB2, _L)])

    rem = (k_sel - cnt_a) - cnt_a2
    tbits = (bstar << 21) | (b2star << 16)
    that = lax.bitcast_convert_type(tbits, jnp.float32)
    neg_sum = sum_a + sum_a2 + jnp.where(rem > 0.0, rem * that, 0.0)
    result = (pos_sum + neg_sum) / (pos_num + k_sel + _EPS)
    out_buf[...] = result

    @pl.when((sid == 0) & (cid == 0))
    def _():
        pltpu.sync_copy(out_buf, out_hbm)


def _sc_select(word1d, stats1d):
    mesh = plsc.VectorSubcoreMesh(core_axis_name="c", subcore_axis_name="s")
    f32 = jnp.float32
    fn = pl.kernel(
        _sc_body,
        out_type=jax.ShapeDtypeStruct((_L,), f32),
        mesh=mesh,
        compiler_params=pltpu.CompilerParams(needs_layout_passes=False),
        scratch_types=[
            pltpu.VMEM((_B * _PER_TILE,), jnp.int32),
            pltpu.VMEM((_HIST,), f32),
            pltpu.VMEM((_ROW,), f32),
            pltpu.VMEM((_ROW,), f32),
            pltpu.VMEM((1024,), f32),
            pltpu.VMEM((_L,), f32),
            pltpu.VMEM_SHARED((_NTILES, _ROW), f32),
            pltpu.SemaphoreType.DMA,
            pltpu.SemaphoreType.DMA,
        ],
    )
    return fn(word1d, stats1d)


def kernel(pred, probMap, probMask):
    pred3 = pred.reshape(_B, 512, 512)
    map3 = probMap.reshape(_B, 512, 512)
    mask3 = probMask.reshape(_B, 512, 512)
    words, stats = _tc_stage(pred3, map3, mask3)
    out16 = _sc_select(words.reshape(-1), stats.reshape(-1))
    return out16[0]


# speedup vs baseline: 1.0197x; 1.0197x over previous
import functools

import jax
import jax.numpy as jnp
from jax import lax
from jax.experimental import pallas as pl
from jax.experimental.pallas import tpu as pltpu
from jax.experimental.pallas import tpu_sc as plsc

_RATIO = 3.0
_EPS = 1e-06

_B = 4
_NPOS = 512 * 512
_NTILES = 16
_PER_TILE = _NPOS // _NTILES
_CH = 8192
_NCH = _PER_TILE // _CH
_NB = 2048
_NB2 = 32
_L = 16
_HIST = _NB * _L
_ROW = _NB
_ROW2 = 128

_GATHER_DNUMS = lax.GatherDimensionNumbers(
    offset_dims=(), collapsed_slice_dims=(0,), start_index_map=(0,))


def _dg(v, idx):
    return lax.gather(v, idx[:, None], _GATHER_DNUMS, (1,),
                      mode=lax.GatherScatterMode.PROMISE_IN_BOUNDS)


_BH = 512
_G = 512 // _BH
_RB = _BH * 512 // 128


def _tc_body(pred_ref, t_ref, msk_ref, word_ref, stats_ref,
             sl_ref, sp_ref, macc_ref, lacc_ref):
    i = pl.program_id(0)
    j = pl.program_id(1)
    p = pred_ref[0]
    t = t_ref[0]
    mk = msk_ref[0]
    x = jnp.where(t > 0.5, p, 1.0 - p)
    loss = jnp.maximum(jnp.minimum(-jnp.log(x), 100.0), 0.0)
    pos = t * mk
    neg = (1.0 - t) * mk
    lacc_ref[j] = loss

    @pl.when(j == 0)
    def _():
        sl_ref[...] = loss
        sp_ref[...] = pos
        macc_ref[...] = neg

    @pl.when(j > 0)
    def _():
        sl_ref[...] = sl_ref[...] + loss
        sp_ref[...] = sp_ref[...] + pos
        macc_ref[...] = macc_ref[...] + neg

    @pl.when((i == 0) & (j == 0))
    def _():
        stats_ref[...] = jnp.zeros_like(stats_ref)

    @pl.when(j == _B - 1)
    def _():
        w = macc_ref[...].astype(jnp.int32)
        for jj in range(_B):
            bits = lax.bitcast_convert_type(lacc_ref[jj], jnp.int32)
            rnd = (bits + 0x8000) & jnp.int32(-65536)
            word_ref[pl.ds(jj * _RB, _RB), :] = (rnd | w).reshape(_RB, 128)
        stats_ref[0:1, :] = stats_ref[0:1, :] + jnp.sum(sp_ref[...])
        stats_ref[1:2, :] = stats_ref[1:2, :] + jnp.sum(macc_ref[...])
        stats_ref[2:3, :] = stats_ref[2:3, :] + jnp.sum(
            sl_ref[...] * sp_ref[...])


def _tc_stage(pred3, map3, mask3):
    in_spec = pl.BlockSpec((1, _BH, 512), lambda i, j: (j, i, 0))
    return pl.pallas_call(
        _tc_body,
        grid=(_G, _B),
        in_specs=[in_spec, in_spec, in_spec],
        out_specs=[
            pl.BlockSpec((_B * _RB, 128), lambda i, j: (i, 0)),
            pl.BlockSpec((8, 128), lambda i, j: (0, 0)),
        ],
        out_shape=[
            jax.ShapeDtypeStruct((_G * _B * _RB, 128), jnp.int32),
            jax.ShapeDtypeStruct((8, 128), jnp.float32),
        ],
        scratch_shapes=[
            pltpu.VMEM((_BH, 512), jnp.float32),
            pltpu.VMEM((_BH, 512), jnp.float32),
            pltpu.VMEM((_BH, 512), jnp.float32),
            pltpu.VMEM((_B, _BH, 512), jnp.float32),
        ],
    )(pred3, map3, mask3)


def _sc_body(word_hbm, stats_hbm, out_hbm,
             vbuf, hist_ref, red_ref, acc_ref,
             stats_buf, out_buf, shared, sem0, sem2):
    sid = lax.axis_index("s")
    cid = lax.axis_index("c")
    base = sid * (_B * _PER_TILE)

    _STRIPE = _B * _PER_TILE
    first_handles = [
        pltpu.async_copy(word_hbm.at[pl.ds(base + h * (_STRIPE // 4),
                                           _STRIPE // 4)],
                         vbuf.at[pl.ds(h * (_STRIPE // 4), _STRIPE // 4)],
                         sem0)
        for h in range(4)]
    pltpu.sync_copy(stats_hbm, stats_buf)
    pos_num = stats_buf[pl.ds(0, _L)]
    neg_count = stats_buf[pl.ds(128, _L)]
    pos_sum = stats_buf[pl.ds(256, _L)]
    k_sel = jnp.minimum(neg_count, pos_num * _RATIO)

    lane = lax.iota(jnp.int32, _L)
    zeros16 = jnp.zeros((_L,), jnp.float32)
    lane15 = jnp.full((_L,), _L - 1, jnp.int32)

    def splat_sum(v):
        for s in (1, 2, 4, 8):
            v = v + _dg(v, lane ^ s)
        return v

    def incl_prefix(v):
        r = v
        for s in (1, 2, 4, 8):
            g = _dg(r, jnp.maximum(lane - s, 0))
            r = r + jnp.where(lane >= s, g, zeros16)
        return r

    def zero_hist(nwords):
        @plsc.parallel_loop(0, nwords // _L, unroll=8)
        def _(i):
            hist_ref[pl.ds(i * _L, _L)] = zeros16

    def hist_pass(bstar_splat, lo, hi):
        @plsc.parallel_loop(lo // _L, hi // _L, unroll=4, carry=zeros16)
        def vb(i, sa):
            bits = vbuf[pl.ds(i * _L, _L)]
            b1 = bits >> 21
            wf = (bits & 7).astype(jnp.float32)
            if bstar_splat is None:
                plsc.addupdate_scatter(hist_ref, [lane * _NB + b1], wf)
            else:
                sel = b1 == bstar_splat
                b2 = (bits >> 16) & (_NB2 - 1)
                idx = lane * _NB2 + b2
                v = lax.bitcast_convert_type(bits & jnp.int32(-65536),
                                             jnp.float32)
                wv = wf * v
                plsc.addupdate_scatter(hist_ref, [idx], wf, mask=sel)
                plsc.addupdate_scatter(hist_ref, [idx + _NB2 * _L], wv,
                                       mask=sel)
                sa = sa + jnp.where(b1 > bstar_splat, wv, zeros16)
            return sa
        return vb

    def lane_reduce(nbins, with_sums):
        @plsc.parallel_loop(0, nbins // _L, unroll=2)
        def _(i):
            cacc = zeros16
            for l in range(_L):
                cacc = cacc + hist_ref[pl.ds(l * nbins + i * _L, _L)]
            red_ref[pl.ds(i * _L, _L)] = cacc
            if with_sums:
                sacc = zeros16
                for l in range(_L):
                    sacc = sacc + hist_ref[
                        pl.ds(nbins * _L + l * nbins + i * _L, _L)]
                red_ref[pl.ds(nbins + i * _L, _L)] = sacc

    def merge(width):
        pltpu.sync_copy(red_ref.at[pl.ds(0, width)],
                        shared.at[sid, pl.ds(0, width)])
        plsc.subcore_barrier()
        hs = [pltpu.async_copy(shared.at[r, pl.ds(0, width)],
                               hist_ref.at[pl.ds(r * width, width)], sem2)
              for r in range(_NTILES)]
        for h in hs:
            h.wait()

        @plsc.parallel_loop(0, width // _L, unroll=2)
        def _(i):
            o = i * _L
            a = hist_ref[pl.ds(o, _L)]
            for r in range(1, _NTILES):
                a = a + hist_ref[pl.ds(r * width + o, _L)]
            acc_ref[pl.ds(o, _L)] = a
        plsc.subcore_barrier()

    def scan(ktarget, nbins, with_sums):
        def tb(i, tot):
            return tot + acc_ref[pl.ds(i * _L, _L)]
        total = splat_sum(lax.fori_loop(0, nbins // _L, tb, zeros16))
        thresh = total - ktarget

        def sb(i, carry):
            run, nm = carry
            v = acc_ref[pl.ds(i * _L, _L)]
            incl = incl_prefix(v)
            cb = run + (incl - v)
            nm = nm + jnp.where(cb <= thresh, 1.0, 0.0)
            return run + _dg(incl, lane15), nm
        _, nmask = lax.fori_loop(0, nbins // _L, sb, (zeros16, zeros16))
        bstar = (splat_sum(nmask) - 1.0).astype(jnp.int32)

        def ab(i, carry):
            ca, sa = carry
            msk = (lane + i * _L) > bstar
            ca = ca + jnp.where(msk, acc_ref[pl.ds(i * _L, _L)], zeros16)
            if with_sums:
                sa = sa + jnp.where(msk, acc_ref[pl.ds(nbins + i * _L, _L)],
                                    zeros16)
            return ca, sa
        ca, sa = lax.fori_loop(0, nbins // _L, ab, (zeros16, zeros16))
        return bstar, splat_sum(ca), splat_sum(sa)

    zero_hist(_HIST)
    for h in range(4):
        first_handles[h].wait()
        hist_pass(None, h * (_STRIPE // 4), (h + 1) * (_STRIPE // 4))
    lane_reduce(_NB, False)
    merge(_ROW)
    bstar, cnt_a, _ = scan(k_sel, _NB, False)

    zero_hist(2 * _NB2 * _L)
    sa_vec = hist_pass(bstar, 0, _STRIPE)
    lane_reduce(_NB2, True)
    red_ref[pl.ds(2 * _NB2, _L)] = sa_vec
    merge(_ROW2)
    b2star, cnt_a2, sum_a2 = scan(k_sel - cnt_a, _NB2, True)
    sum_a = splat_sum(acc_ref[pl.ds(2 * _NB2, _L)])

    rem = (k_sel - cnt_a) - cnt_a2
    tbits = (bstar << 21) | (b2star << 16)
    that = lax.bitcast_convert_type(tbits, jnp.float32)
    neg_sum = sum_a + sum_a2 + jnp.where(rem > 0.0, rem * that, 0.0)
    result = (pos_sum + neg_sum) / (pos_num + k_sel + _EPS)
    out_buf[...] = result

    @pl.when((sid == 0) & (cid == 0))
    def _():
        pltpu.sync_copy(out_buf, out_hbm)


def _sc_select(word1d, stats1d):
    mesh = plsc.VectorSubcoreMesh(core_axis_name="c", subcore_axis_name="s")
    f32 = jnp.float32
    fn = pl.kernel(
        _sc_body,
        out_type=jax.ShapeDtypeStruct((_L,), f32),
        mesh=mesh,
        compiler_params=pltpu.CompilerParams(needs_layout_passes=False),
        scratch_types=[
            pltpu.VMEM((_B * _PER_TILE,), jnp.int32),
            pltpu.VMEM((_HIST,), f32),
            pltpu.VMEM((_ROW,), f32),
            pltpu.VMEM((_ROW,), f32),
            pltpu.VMEM((1024,), f32),
            pltpu.VMEM((_L,), f32),
            pltpu.VMEM_SHARED((_NTILES, _ROW), f32),
            pltpu.SemaphoreType.DMA,
            pltpu.SemaphoreType.DMA,
        ],
    )
    return fn(word1d, stats1d)


def kernel(pred, probMap, probMask):
    pred3 = pred.reshape(_B, 512, 512)
    map3 = probMap.reshape(_B, 512, 512)
    mask3 = probMask.reshape(_B, 512, 512)
    words, stats = _tc_stage(pred3, map3, mask3)
    out16 = _sc_select(words.reshape(-1), stats.reshape(-1))
    return out16[0]


# speedup vs baseline: 1.0204x; 1.0007x over previous
import jax
import jax.numpy as jnp
from jax import lax
from jax.experimental import pallas as pl
from jax.experimental.pallas import tpu as pltpu
from jax.experimental.pallas import tpu_sc as plsc

_RATIO = 3.0
_EPS = 1e-06

_B = 4
_NPOS = 512 * 512
_NTILES = 16
_PER_TILE = _NPOS // _NTILES
_NB = 2048
_NB2 = 32
_L = 16
_HIST = _NB * _L
_ROW = _NB
_ROW2 = 128

_GATHER_DNUMS = lax.GatherDimensionNumbers(
    offset_dims=(), collapsed_slice_dims=(0,), start_index_map=(0,))


def _dg(v, idx):
    return lax.gather(v, idx[:, None], _GATHER_DNUMS, (1,),
                      mode=lax.GatherScatterMode.PROMISE_IN_BOUNDS)


_BH = 512
_G = 512 // _BH
_RB = _BH * 512 // 128


def _tc_body(pred_ref, t_ref, msk_ref, word_ref, stats_ref,
             sl_ref, sp_ref, macc_ref, lacc_ref):
    i = pl.program_id(0)
    j = pl.program_id(1)
    p = pred_ref[0]
    t = t_ref[0]
    mk = msk_ref[0]
    x = jnp.where(t > 0.5, p, 1.0 - p)
    loss = jnp.maximum(jnp.minimum(-jnp.log(x), 100.0), 0.0)
    pos = t * mk
    neg = (1.0 - t) * mk
    lacc_ref[j] = loss

    @pl.when(j == 0)
    def _():
        sl_ref[...] = loss
        sp_ref[...] = pos
        macc_ref[...] = neg

    @pl.when(j > 0)
    def _():
        sl_ref[...] = sl_ref[...] + loss
        sp_ref[...] = sp_ref[...] + pos
        macc_ref[...] = macc_ref[...] + neg

    @pl.when((i == 0) & (j == 0))
    def _():
        stats_ref[...] = jnp.zeros_like(stats_ref)

    @pl.when(j == _B - 1)
    def _():
        w = macc_ref[...].astype(jnp.int32)
        for jj in range(_B):
            bits = lax.bitcast_convert_type(lacc_ref[jj], jnp.int32)
            rnd = (bits + 0x8000) & jnp.int32(-65536)
            word_ref[pl.ds(jj * _RB, _RB), :] = (rnd | w).reshape(_RB, 128)
        stats_ref[0:1, :] = stats_ref[0:1, :] + jnp.sum(sp_ref[...])
        stats_ref[1:2, :] = stats_ref[1:2, :] + jnp.sum(macc_ref[...])
        stats_ref[2:3, :] = stats_ref[2:3, :] + jnp.sum(
            sl_ref[...] * sp_ref[...])


def _tc_stage(pred3, map3, mask3):
    in_spec = pl.BlockSpec((1, _BH, 512), lambda i, j: (j, i, 0))
    return pl.pallas_call(
        _tc_body,
        grid=(_G, _B),
        in_specs=[in_spec, in_spec, in_spec],
        out_specs=[
            pl.BlockSpec((_B * _RB, 128), lambda i, j: (i, 0)),
            pl.BlockSpec((8, 128), lambda i, j: (0, 0)),
        ],
        out_shape=[
            jax.ShapeDtypeStruct((_G * _B * _RB, 128), jnp.int32),
            jax.ShapeDtypeStruct((8, 128), jnp.float32),
        ],
        scratch_shapes=[
            pltpu.VMEM((_BH, 512), jnp.float32),
            pltpu.VMEM((_BH, 512), jnp.float32),
            pltpu.VMEM((_BH, 512), jnp.float32),
            pltpu.VMEM((_B, _BH, 512), jnp.float32),
        ],
    )(pred3, map3, mask3)


def _sc_body(word_hbm, stats_hbm, out_hbm,
             vbuf, hist_ref, red_ref, acc_ref,
             stats_buf, out_buf, shared, sem0, sem2):
    sid = lax.axis_index("s")
    cid = lax.axis_index("c")
    base = sid * (_B * _PER_TILE)

    _STRIPE = _B * _PER_TILE
    first_handles = [
        pltpu.async_copy(word_hbm.at[pl.ds(base + h * (_STRIPE // 4),
                                           _STRIPE // 4)],
                         vbuf.at[pl.ds(h * (_STRIPE // 4), _STRIPE // 4)],
                         sem0)
        for h in range(4)]
    pltpu.sync_copy(stats_hbm, stats_buf)
    pos_num = stats_buf[pl.ds(0, _L)]
    neg_count = stats_buf[pl.ds(128, _L)]
    pos_sum = stats_buf[pl.ds(256, _L)]
    k_sel = jnp.minimum(neg_count, pos_num * _RATIO)

    lane = lax.iota(jnp.int32, _L)
    zeros16 = jnp.zeros((_L,), jnp.float32)
    lane15 = jnp.full((_L,), _L - 1, jnp.int32)

    def splat_sum(v):
        for s in (1, 2, 4, 8):
            v = v + _dg(v, lane ^ s)
        return v

    def incl_prefix(v):
        r = v
        for s in (1, 2, 4, 8):
            g = _dg(r, jnp.maximum(lane - s, 0))
            r = r + jnp.where(lane >= s, g, zeros16)
        return r

    def zero_hist(nwords):
        @plsc.parallel_loop(0, nwords // _L, unroll=8)
        def _(i):
            hist_ref[pl.ds(i * _L, _L)] = zeros16

    def hist_pass(bstar_splat, lo, hi):
        @plsc.parallel_loop(lo // _L, hi // _L, unroll=4, carry=zeros16)
        def vb(i, sa):
            bits = vbuf[pl.ds(i * _L, _L)]
            b1 = bits >> 21
            wf = (bits & 7).astype(jnp.float32)
            if bstar_splat is None:
                plsc.addupdate_scatter(hist_ref, [lane * _NB + b1], wf)
            else:
                sel = b1 == bstar_splat
                b2 = (bits >> 16) & (_NB2 - 1)
                idx = lane * _NB2 + b2
                v = lax.bitcast_convert_type(bits & jnp.int32(-65536),
                                             jnp.float32)
                wv = wf * v
                plsc.addupdate_scatter(hist_ref, [idx], wf, mask=sel)
                plsc.addupdate_scatter(hist_ref, [idx + _NB2 * _L], wv,
                                       mask=sel)
                sa = sa + jnp.where(b1 > bstar_splat, wv, zeros16)
            return sa
        return vb

    def lane_reduce(nbins, with_sums):
        @plsc.parallel_loop(0, nbins // _L, unroll=2)
        def _(i):
            cacc = zeros16
            for l in range(_L):
                cacc = cacc + hist_ref[pl.ds(l * nbins + i * _L, _L)]
            red_ref[pl.ds(i * _L, _L)] = cacc
            if with_sums:
                sacc = zeros16
                for l in range(_L):
                    sacc = sacc + hist_ref[
                        pl.ds(nbins * _L + l * nbins + i * _L, _L)]
                red_ref[pl.ds(nbins + i * _L, _L)] = sacc

    def merge(width):
        pltpu.sync_copy(red_ref.at[pl.ds(0, width)],
                        shared.at[sid, pl.ds(0, width)])
        plsc.subcore_barrier()
        hs = [pltpu.async_copy(shared.at[r, pl.ds(0, width)],
                               hist_ref.at[pl.ds(r * width, width)], sem2)
              for r in range(_NTILES)]
        for h in hs:
            h.wait()

        @plsc.parallel_loop(0, width // _L, unroll=2)
        def _(i):
            o = i * _L
            a = hist_ref[pl.ds(o, _L)]
            for r in range(1, _NTILES):
                a = a + hist_ref[pl.ds(r * width + o, _L)]
            acc_ref[pl.ds(o, _L)] = a
        plsc.subcore_barrier()

    def scan(ktarget, nbins, with_sums):
        def tb(i, tot):
            return tot + acc_ref[pl.ds(i * _L, _L)]
        total = splat_sum(lax.fori_loop(0, nbins // _L, tb, zeros16))
        thresh = total - ktarget

        def sb(i, carry):
            run, nm = carry
            v = acc_ref[pl.ds(i * _L, _L)]
            incl = incl_prefix(v)
            cb = run + (incl - v)
            nm = nm + jnp.where(cb <= thresh, 1.0, 0.0)
            return run + _dg(incl, lane15), nm
        _, nmask = lax.fori_loop(0, nbins // _L, sb, (zeros16, zeros16))
        bstar = (splat_sum(nmask) - 1.0).astype(jnp.int32)

        def ab(i, carry):
            ca, sa = carry
            msk = (lane + i * _L) > bstar
            ca = ca + jnp.where(msk, acc_ref[pl.ds(i * _L, _L)], zeros16)
            if with_sums:
                sa = sa + jnp.where(msk, acc_ref[pl.ds(nbins + i * _L, _L)],
                                    zeros16)
            return ca, sa
        ca, sa = lax.fori_loop(0, nbins // _L, ab, (zeros16, zeros16))
        return bstar, splat_sum(ca), splat_sum(sa)

    zero_hist(_HIST)
    for h in range(4):
        first_handles[h].wait()
        hist_pass(None, h * (_STRIPE // 4), (h + 1) * (_STRIPE // 4))
    lane_reduce(_NB, False)
    merge(_ROW)
    bstar, cnt_a, _ = scan(k_sel, _NB, False)

    zero_hist(2 * _NB2 * _L)
    sa_vec = hist_pass(bstar, 0, _STRIPE)
    lane_reduce(_NB2, True)
    red_ref[pl.ds(2 * _NB2, _L)] = sa_vec
    merge(_ROW2)
    b2star, cnt_a2, sum_a2 = scan(k_sel - cnt_a, _NB2, True)
    sum_a = splat_sum(acc_ref[pl.ds(2 * _NB2, _L)])

    rem = (k_sel - cnt_a) - cnt_a2
    tbits = (bstar << 21) | (b2star << 16)
    that = lax.bitcast_convert_type(tbits, jnp.float32)
    neg_sum = sum_a + sum_a2 + jnp.where(rem > 0.0, rem * that, 0.0)
    result = (pos_sum + neg_sum) / (pos_num + k_sel + _EPS)
    out_buf[...] = result

    @pl.when((sid == 0) & (cid == 0))
    def _():
        pltpu.sync_copy(out_buf, out_hbm)


def _sc_select(word1d, stats1d):
    mesh = plsc.VectorSubcoreMesh(core_axis_name="c", subcore_axis_name="s")
    f32 = jnp.float32
    fn = pl.kernel(
        _sc_body,
        out_type=jax.ShapeDtypeStruct((_L,), f32),
        mesh=mesh,
        compiler_params=pltpu.CompilerParams(needs_layout_passes=False),
        scratch_types=[
            pltpu.VMEM((_B * _PER_TILE,), jnp.int32),
            pltpu.VMEM((_HIST,), f32),
            pltpu.VMEM((_ROW,), f32),
            pltpu.VMEM((_ROW,), f32),
            pltpu.VMEM((1024,), f32),
            pltpu.VMEM((_L,), f32),
            pltpu.VMEM_SHARED((_NTILES, _ROW), f32),
            pltpu.SemaphoreType.DMA,
            pltpu.SemaphoreType.DMA,
        ],
    )
    return fn(word1d, stats1d)


def kernel(pred, probMap, probMask):
    pred3 = pred.reshape(_B, 512, 512)
    map3 = probMap.reshape(_B, 512, 512)
    mask3 = probMask.reshape(_B, 512, 512)
    words, stats = _tc_stage(pred3, map3, mask3)
    out16 = _sc_select(words.reshape(-1), stats.reshape(-1))
    return out16[0]
